# baseline (speedup 1.0000x reference)
import functools

import jax
import jax.numpy as jnp
from jax import lax
from jax.experimental import pallas as pl
from jax.experimental.pallas import tpu as pltpu
from jax.experimental.pallas import tpu_sc as plsc

LEAKY = 0.2
EPS = 1e-5
E_REAL = 10000
E_PAD = 10240
N_WORKERS = 32
E_PER_W = E_PAD // N_WORKERS
CHUNK = 32
N_CHUNKS = E_PER_W // CHUNK
E_TILE = 256


def _sc_gather(xT, idxR):
    C = xT.shape[1]
    dt = xT.dtype
    n_gchunks = E_PAD // CHUNK
    mesh = plsc.VectorSubcoreMesh(core_axis_name="c", subcore_axis_name="s")

    @functools.partial(
        pl.kernel,
        mesh=mesh,
        out_type=jax.ShapeDtypeStruct((n_gchunks, 4 * CHUNK, C), dt),
        scratch_types=(
            [pltpu.VMEM((N_CHUNKS, 4 * CHUNK), jnp.int32)]
            + [pltpu.VMEM((4 * CHUNK, C), dt) for _ in range(2)]
            + [pltpu.SemaphoreType.DMA for _ in range(4)]
        ),
    )
    def k(x_hbm, idx_hbm, out,
          idx_v, b0, b1, g0, g1, w0, w1):
        bufs = (b0, b1)
        gsems = (g0, g1)
        wsems = (w0, w1)
        NB = 2
        wid = lax.axis_index("s") * 2 + lax.axis_index("c")
        gbase = wid * N_CHUNKS
        pltpu.sync_copy(idx_hbm.at[wid], idx_v)

        gcp = [None] * N_CHUNKS
        wcp = [None] * N_CHUNKS
        for c in range(NB - 1):
            gcp[c] = pltpu.async_copy(x_hbm.at[idx_v.at[c]],
                                      bufs[c % NB], gsems[c % NB])
        for c in range(N_CHUNKS):
            p = c % NB
            if c + NB - 1 < N_CHUNKS:
                q = (c + NB - 1) % NB
                if c >= 1:
                    wcp[c - 1].wait()
                gcp[c + NB - 1] = pltpu.async_copy(
                    x_hbm.at[idx_v.at[c + NB - 1]], bufs[q], gsems[q])
            gcp[c].wait()
            wcp[c] = pltpu.async_copy(bufs[p], out.at[gbase + c], wsems[p])
        for c in range(max(0, N_CHUNKS - NB), N_CHUNKS):
            wcp[c].wait()

    return k(xT, idxR)


def _tc_conv(x_own, gath, W5, b, out_dtype):
    C = x_own.shape[1]
    O = W5.shape[1]
    n_tiles = E_PAD // E_TILE
    cpt = E_TILE // CHUNK

    def body(x_ref, g1_ref, g2_ref, g3_ref, g4_ref, w_ref, b_ref,
             y_ref, st_ref):
        i = pl.program_id(0)

        @pl.when(i == 0)
        def _():
            st_ref[...] = jnp.zeros_like(st_ref)

        f0 = x_ref[...]
        a1 = g1_ref[...].reshape(E_TILE, C).astype(jnp.float32)
        a2 = g2_ref[...].reshape(E_TILE, C).astype(jnp.float32)
        a3 = g3_ref[...].reshape(E_TILE, C).astype(jnp.float32)
        a4 = g4_ref[...].reshape(E_TILE, C).astype(jnp.float32)
        G = jnp.concatenate(
            [f0.astype(jnp.float32), a1 + a3, a2 + a4,
             jnp.abs(a1 - a3), jnp.abs(a2 - a4)],
            axis=1).astype(jnp.bfloat16)
        y = jnp.dot(G, w_ref[...].astype(jnp.bfloat16),
                    preferred_element_type=jnp.float32)
        y = y + b_ref[...]
        y = jnp.where(y >= 0.0, y, LEAKY * y)
        y_ref[...] = y.astype(y_ref.dtype)

        row = i * E_TILE + lax.broadcasted_iota(jnp.int32, (E_TILE, 1), 0)
        ym = jnp.where(row < E_REAL, y, 0.0)
        s = jnp.sum(ym, axis=0, keepdims=True)
        q = jnp.sum(ym * ym, axis=0, keepdims=True)
        st_ref[0:1, :] += s
        st_ref[1:2, :] += q

    y, st = pl.pallas_call(
        body,
        grid=(n_tiles,),
        in_specs=[
            pl.BlockSpec((E_TILE, C), lambda i: (i, 0)),
            pl.BlockSpec((cpt, CHUNK, C), lambda i: (i, 0, 0)),
            pl.BlockSpec((cpt, CHUNK, C), lambda i: (i, 1, 0)),
            pl.BlockSpec((cpt, CHUNK, C), lambda i: (i, 2, 0)),
            pl.BlockSpec((cpt, CHUNK, C), lambda i: (i, 3, 0)),
            pl.BlockSpec((5 * C, O), lambda i: (0, 0)),
            pl.BlockSpec((1, O), lambda i: (0, 0)),
        ],
        out_specs=[
            pl.BlockSpec((E_TILE, O), lambda i: (i, 0)),
            pl.BlockSpec((8, O), lambda i: (0, 0)),
        ],
        out_shape=[
            jax.ShapeDtypeStruct((E_PAD, O), out_dtype),
            jax.ShapeDtypeStruct((8, O), jnp.float32),
        ],
    )(x_own, gath, gath, gath, gath, W5, b)
    return y, st


def _tc_affine(y, scale, shift):
    O = y.shape[1]
    n_tiles = E_PAD // E_TILE

    def body(y_ref, sc_ref, sh_ref, o_ref):
        o_ref[...] = y_ref[...] * sc_ref[...] + sh_ref[...]

    return pl.pallas_call(
        body,
        grid=(n_tiles,),
        in_specs=[
            pl.BlockSpec((E_TILE, O), lambda i: (i, 0)),
            pl.BlockSpec((1, O), lambda i: (0, 0)),
            pl.BlockSpec((1, O), lambda i: (0, 0)),
        ],
        out_specs=pl.BlockSpec((E_TILE, O), lambda i: (i, 0)),
        out_shape=jax.ShapeDtypeStruct((E_PAD, O), jnp.float32),
    )(y, scale, shift)


def _norm_affine(st):
    s = st[0]
    q = st[1]
    m = s / E_REAL
    v = q / E_REAL - m * m
    inv = 1.0 / jnp.sqrt(v + EPS)
    S = (1.0 + 1.0 / jnp.sqrt(v / (v + EPS) + EPS)) * inv
    return S[None, :], (-m * S)[None, :]


def _w5(W):
    return jnp.transpose(W, (2, 1, 0)).reshape(-1, W.shape[0])


def _as_i32(x_bf16):
    s = x_bf16.shape
    return lax.bitcast_convert_type(
        x_bf16.reshape(s[:-1] + (s[-1] // 2, 2)), jnp.int32)


def _as_bf16(x_i32):
    s = x_i32.shape
    return lax.bitcast_convert_type(x_i32, jnp.bfloat16).reshape(
        s[:-1] + (2 * s[-1],))


def kernel(in_x, gemm_edges, W1_0, b1_0, W2_0, b2_0, W1_1, b1_1, W2_1, b2_1):
    xT = jnp.transpose(in_x[0])
    xT = jnp.pad(xT, ((0, E_PAD - E_REAL), (0, 0)))
    idx = gemm_edges[0].astype(jnp.int32)
    idx4 = jnp.pad(jnp.transpose(idx), ((0, 0), (0, E_PAD - E_REAL)))
    idxR = jnp.transpose(
        idx4.reshape(4, N_WORKERS, N_CHUNKS, CHUNK),
        (1, 2, 0, 3)).reshape(N_WORKERS, N_CHUNKS, 4 * CHUNK)

    g0 = _sc_gather(xT, idxR)
    W5_0 = _w5(W1_0)
    y0, st0 = _tc_conv(xT, g0, W5_0, b1_0[None, :], jnp.bfloat16)
    S0, t0 = _norm_affine(st0)

    W1f = W1_1 * S0[0][None, :, None]
    bf = b1_1 + (W1_1[:, :, 0] + 2.0 * W1_1[:, :, 1]
                 + 2.0 * W1_1[:, :, 2]) @ t0[0]

    g1 = _as_bf16(_sc_gather(_as_i32(y0), idxR))
    W5_1 = _w5(W1f)
    y1, st1 = _tc_conv(y0, g1, W5_1, bf[None, :], jnp.float32)
    S1, t1 = _norm_affine(st1)

    out = _tc_affine(y1, S1, t1)
    return jnp.transpose(out[:E_REAL])[None]


# speedup vs baseline: 1.6046x; 1.6046x over previous
import functools

import jax
import jax.numpy as jnp
from jax import lax
from jax.experimental import pallas as pl
from jax.experimental.pallas import tpu as pltpu
from jax.experimental.pallas import tpu_sc as plsc

LEAKY = 0.2
EPS = 1e-5
E_REAL = 10000
E_PAD = 10240
N_WORKERS = 32
E_PER_W = E_PAD // N_WORKERS
CHUNK = 32
N_CHUNKS = E_PER_W // CHUNK
E_TILE = 256


def _sc_gather(xT, idxR):
    C = xT.shape[1]
    n_gchunks = E_PAD // CHUNK
    mesh = plsc.VectorSubcoreMesh(core_axis_name="c", subcore_axis_name="s")

    @functools.partial(
        pl.kernel,
        mesh=mesh,
        out_type=jax.ShapeDtypeStruct((n_gchunks, 4 * CHUNK, C),
                                      jnp.float32),
        scratch_types=(
            [pltpu.VMEM((N_CHUNKS, 4 * CHUNK), jnp.int32)]
            + [pltpu.VMEM((4 * CHUNK, C), jnp.float32) for _ in range(2)]
            + [pltpu.SemaphoreType.DMA for _ in range(4)]
        ),
    )
    def k(x_hbm, idx_hbm, out,
          idx_v, b0, b1, g0, g1, w0, w1):
        bufs = (b0, b1)
        gsems = (g0, g1)
        wsems = (w0, w1)
        NB = 2
        wid = lax.axis_index("s") * 2 + lax.axis_index("c")
        gbase = wid * N_CHUNKS
        pltpu.sync_copy(idx_hbm.at[wid], idx_v)

        gcp = [None] * N_CHUNKS
        wcp = [None] * N_CHUNKS
        for c in range(NB - 1):
            gcp[c] = pltpu.async_copy(x_hbm.at[idx_v.at[c]],
                                      bufs[c % NB], gsems[c % NB])
        for c in range(N_CHUNKS):
            p = c % NB
            if c + NB - 1 < N_CHUNKS:
                q = (c + NB - 1) % NB
                if c >= 1:
                    wcp[c - 1].wait()
                gcp[c + NB - 1] = pltpu.async_copy(
                    x_hbm.at[idx_v.at[c + NB - 1]], bufs[q], gsems[q])
            gcp[c].wait()
            wcp[c] = pltpu.async_copy(bufs[p], out.at[gbase + c], wsems[p])
        for c in range(max(0, N_CHUNKS - NB), N_CHUNKS):
            wcp[c].wait()

    return k(xT, idxR)


def _tc_conv(x_own, gath, W5, b):
    C = x_own.shape[1]
    O = W5.shape[1]
    n_tiles = E_PAD // E_TILE
    cpt = E_TILE // CHUNK

    def body(x_ref, g1_ref, g2_ref, g3_ref, g4_ref, w_ref, b_ref,
             y_ref, st_ref):
        i = pl.program_id(0)

        @pl.when(i == 0)
        def _():
            st_ref[...] = jnp.zeros_like(st_ref)

        f0 = x_ref[...]
        a1 = g1_ref[...].reshape(E_TILE, C)
        a2 = g2_ref[...].reshape(E_TILE, C)
        a3 = g3_ref[...].reshape(E_TILE, C)
        a4 = g4_ref[...].reshape(E_TILE, C)
        G = jnp.concatenate(
            [f0, a1 + a3, a2 + a4, jnp.abs(a1 - a3), jnp.abs(a2 - a4)],
            axis=1)
        y = jnp.dot(G, w_ref[...], preferred_element_type=jnp.float32)
        y = y + b_ref[...]
        y = jnp.where(y >= 0.0, y, LEAKY * y)
        y_ref[...] = y

        row = i * E_TILE + lax.broadcasted_iota(jnp.int32, (E_TILE, 1), 0)
        ym = jnp.where(row < E_REAL, y, 0.0)
        s = jnp.sum(ym, axis=0, keepdims=True)
        q = jnp.sum(ym * ym, axis=0, keepdims=True)
        st_ref[0:1, :] += s
        st_ref[1:2, :] += q

    y, st = pl.pallas_call(
        body,
        grid=(n_tiles,),
        in_specs=[
            pl.BlockSpec((E_TILE, C), lambda i: (i, 0)),
            pl.BlockSpec((cpt, CHUNK, C), lambda i: (i, 0, 0)),
            pl.BlockSpec((cpt, CHUNK, C), lambda i: (i, 1, 0)),
            pl.BlockSpec((cpt, CHUNK, C), lambda i: (i, 2, 0)),
            pl.BlockSpec((cpt, CHUNK, C), lambda i: (i, 3, 0)),
            pl.BlockSpec((5 * C, O), lambda i: (0, 0)),
            pl.BlockSpec((1, O), lambda i: (0, 0)),
        ],
        out_specs=[
            pl.BlockSpec((E_TILE, O), lambda i: (i, 0)),
            pl.BlockSpec((8, O), lambda i: (0, 0)),
        ],
        out_shape=[
            jax.ShapeDtypeStruct((E_PAD, O), jnp.float32),
            jax.ShapeDtypeStruct((8, O), jnp.float32),
        ],
    )(x_own, gath, gath, gath, gath, W5, b)
    return y, st


def _tc_affine(y, scale, shift):
    O = y.shape[1]
    n_tiles = E_PAD // E_TILE

    def body(y_ref, sc_ref, sh_ref, o_ref):
        o_ref[...] = y_ref[...] * sc_ref[...] + sh_ref[...]

    return pl.pallas_call(
        body,
        grid=(n_tiles,),
        in_specs=[
            pl.BlockSpec((E_TILE, O), lambda i: (i, 0)),
            pl.BlockSpec((1, O), lambda i: (0, 0)),
            pl.BlockSpec((1, O), lambda i: (0, 0)),
        ],
        out_specs=pl.BlockSpec((E_TILE, O), lambda i: (i, 0)),
        out_shape=jax.ShapeDtypeStruct((E_PAD, O), jnp.float32),
    )(y, scale, shift)


def _norm_affine(st):
    s = st[0]
    q = st[1]
    m = s / E_REAL
    v = q / E_REAL - m * m
    inv = 1.0 / jnp.sqrt(v + EPS)
    S = (1.0 + 1.0 / jnp.sqrt(v / (v + EPS) + EPS)) * inv
    return S[None, :], (-m * S)[None, :]


def _w5(W):
    return jnp.transpose(W, (2, 1, 0)).reshape(-1, W.shape[0])


def kernel(in_x, gemm_edges, W1_0, b1_0, W2_0, b2_0, W1_1, b1_1, W2_1, b2_1):
    xT = jnp.transpose(in_x[0])
    xT = jnp.pad(xT, ((0, E_PAD - E_REAL), (0, 0)))
    idx = gemm_edges[0].astype(jnp.int32)
    idx4 = jnp.pad(jnp.transpose(idx), ((0, 0), (0, E_PAD - E_REAL)))
    idxR = jnp.transpose(
        idx4.reshape(4, N_WORKERS, N_CHUNKS, CHUNK),
        (1, 2, 0, 3)).reshape(N_WORKERS, N_CHUNKS, 4 * CHUNK)

    g0 = _sc_gather(xT, idxR)
    W5_0 = _w5(W1_0)
    y0, st0 = _tc_conv(xT, g0, W5_0, b1_0[None, :])
    S0, t0 = _norm_affine(st0)

    W1f = W1_1 * S0[0][None, :, None]
    bf = b1_1 + (W1_1[:, :, 0] + 2.0 * W1_1[:, :, 1]
                 + 2.0 * W1_1[:, :, 2]) @ t0[0]

    g1 = _sc_gather(y0, idxR)
    W5_1 = _w5(W1f)
    y1, st1 = _tc_conv(y0, g1, W5_1, bf[None, :])
    S1, t1 = _norm_affine(st1)

    out = _tc_affine(y1, S1, t1)
    return jnp.transpose(out[:E_REAL])[None]
